# baseline (speedup 1.0000x reference)
import functools

import jax
import jax.numpy as jnp
from jax import lax
from jax.experimental import pallas as pl
from jax.experimental.pallas import tpu as pltpu
from jax.experimental.pallas import tpu_sc as plsc


def _make_ebd(N, L, V, D, n_cores, n_subcores):
    NW = n_cores * n_subcores
    n_per_w = N // NW
    CH = 32
    n_ch = n_per_w // CH
    scale = float(D) ** 0.5
    LANES = 16

    mesh = plsc.VectorSubcoreMesh(core_axis_name="c", subcore_axis_name="s")

    @functools.partial(
        pl.kernel,
        mesh=mesh,
        out_type=jax.ShapeDtypeStruct((N, D), jnp.float32),
        scratch_types=[
            pltpu.VMEM((n_per_w,), jnp.int32),
            pltpu.VMEM((CH, D), jnp.float32),
            pltpu.VMEM((CH, D), jnp.float32),
            pltpu.SemaphoreType.DMA,
            pltpu.SemaphoreType.DMA,
        ],
    )
    def ebd(table, idx_hbm, pos_hbm, out, idx_v, rows_v, pos_v, gsem, psem):
        wid = lax.axis_index("s") * n_cores + lax.axis_index("c")
        base = wid * n_per_w
        pos_base = lax.rem(base, L)

        pltpu.sync_copy(idx_hbm.at[pl.ds(base, n_per_w)], idx_v)

        def chunk_body(ci, carry):
            b0 = ci * CH
            gather = pltpu.async_copy(
                table.at[idx_v.at[pl.ds(b0, CH)]], rows_v, gsem)
            pcopy = pltpu.async_copy(
                pos_hbm.at[pl.ds(pos_base + b0, CH)], pos_v, psem)
            gather.wait()
            pcopy.wait()

            def row_body(r, c2):
                def vec_body(j, c3):
                    sl = pl.ds(j * LANES, LANES)
                    rows_v[r, sl] = rows_v[r, sl] * scale + pos_v[r, sl]
                    return c3
                return lax.fori_loop(0, D // LANES, vec_body, c2)

            lax.fori_loop(0, CH, row_body, 0)
            pltpu.sync_copy(rows_v, out.at[pl.ds(base + b0, CH)])
            return carry

        lax.fori_loop(0, n_ch, chunk_body, 0)

    return ebd


def kernel(X, word_emb, pos_emb):
    B, L = X.shape
    V, D = word_emb.shape
    N = B * L
    info = plsc.get_sparse_core_info()
    ebd = _make_ebd(N, L, V, D, info.num_cores, info.num_subcores)
    Xf = X.reshape(N).astype(jnp.int32)
    out = ebd(word_emb, Xf, pos_emb[:L])
    return out.reshape(B, L, D)


# speedup vs baseline: 1.7049x; 1.7049x over previous
import functools

import jax
import jax.numpy as jnp
from jax import lax
from jax.experimental import pallas as pl
from jax.experimental.pallas import tpu as pltpu
from jax.experimental.pallas import tpu_sc as plsc


def _make_ebd(N, L, V, D, n_cores, n_subcores):
    NW = n_cores * n_subcores
    n_per_w = N // NW
    CH = 32
    n_ch = n_per_w // CH
    scale = float(D) ** 0.5
    LANES = 16

    mesh = plsc.VectorSubcoreMesh(core_axis_name="c", subcore_axis_name="s")

    @functools.partial(
        pl.kernel,
        mesh=mesh,
        out_type=jax.ShapeDtypeStruct((N, D), jnp.float32),
        scratch_types=[
            pltpu.VMEM((n_per_w,), jnp.int32),
            pltpu.VMEM((CH, D), jnp.float32),
            pltpu.VMEM((CH, D), jnp.float32),
            pltpu.SemaphoreType.DMA,
            pltpu.SemaphoreType.DMA,
        ],
    )
    def ebd(table, idx_hbm, pos_hbm, out, idx_v, rows_v, pos_v, gsem, psem):
        wid = lax.axis_index("s") * n_cores + lax.axis_index("c")
        base = wid * n_per_w
        pos_base = lax.rem(base, L)

        pltpu.sync_copy(idx_hbm.at[pl.ds(base, n_per_w)], idx_v)

        def chunk_body(ci, carry):
            b0 = ci * CH
            gather = pltpu.async_copy(
                table.at[idx_v.at[pl.ds(b0, CH)]], rows_v, gsem)
            pcopy = pltpu.async_copy(
                pos_hbm.at[pl.ds(pos_base + b0, CH)], pos_v, psem)
            gather.wait()
            pcopy.wait()

            def row_body(r, c2):
                for j in range(D // LANES):
                    sl = pl.ds(j * LANES, LANES)
                    rows_v[r, sl] = rows_v[r, sl] * scale + pos_v[r, sl]
                return c2

            lax.fori_loop(0, CH, row_body, 0)
            pltpu.sync_copy(rows_v, out.at[pl.ds(base + b0, CH)])
            return carry

        lax.fori_loop(0, n_ch, chunk_body, 0)

    return ebd


def kernel(X, word_emb, pos_emb):
    B, L = X.shape
    V, D = word_emb.shape
    N = B * L
    info = plsc.get_sparse_core_info()
    ebd = _make_ebd(N, L, V, D, info.num_cores, info.num_subcores)
    Xf = X.reshape(N).astype(jnp.int32)
    out = ebd(word_emb, Xf, pos_emb[:L])
    return out.reshape(B, L, D)


# speedup vs baseline: 2.0500x; 1.2024x over previous
import functools

import jax
import jax.numpy as jnp
from jax import lax
from jax.experimental import pallas as pl
from jax.experimental.pallas import tpu as pltpu
from jax.experimental.pallas import tpu_sc as plsc

LANES = 16
CH = 16
NBUF = 2


def _make_ebd(N, L, V, D, n_cores, n_subcores):
    NW = n_cores * n_subcores
    n_per_w = N // NW
    n_ch = n_per_w // CH
    n_g = n_ch // NBUF
    scale = float(D) ** 0.5

    mesh = plsc.VectorSubcoreMesh(core_axis_name="c", subcore_axis_name="s")

    @functools.partial(
        pl.kernel,
        mesh=mesh,
        out_type=jax.ShapeDtypeStruct((N, D), jnp.float32),
        scratch_types=[
            pltpu.VMEM((n_per_w,), jnp.int32),
            pltpu.VMEM((NBUF, CH, D), jnp.float32),
            pltpu.VMEM((NBUF, CH, D), jnp.float32),
            pltpu.VMEM((NBUF, CH, D), jnp.float32),
        ] + [pltpu.SemaphoreType.DMA] * (3 * NBUF),
    )
    def ebd(table, idx_hbm, pos_hbm, out,
            idx_v, rows_v, pos_v, obuf_v, *sems):
        gsem = sems[0:NBUF]
        psem = sems[NBUF:2 * NBUF]
        ssem = sems[2 * NBUF:3 * NBUF]

        wid = lax.axis_index("s") * n_cores + lax.axis_index("c")
        base = wid * n_per_w
        pos_base = lax.rem(base, L)

        pltpu.sync_copy(idx_hbm.at[pl.ds(base, n_per_w)], idx_v)

        def issue(ci, b):
            pltpu.async_copy(
                table.at[idx_v.at[pl.ds(ci * CH, CH)]], rows_v.at[b], gsem[b])
            pltpu.async_copy(
                pos_hbm.at[pl.ds(pos_base + ci * CH, CH)], pos_v.at[b], psem[b])

        for b in range(NBUF):
            issue(b, b)

        def group(g, carry):
            for b in range(NBUF):
                ci = g * NBUF + b
                pltpu.make_async_copy(
                    table.at[idx_v.at[pl.ds(0, CH)]],
                    rows_v.at[b], gsem[b]).wait()
                pltpu.make_async_copy(
                    pos_hbm.at[pl.ds(0, CH)], pos_v.at[b], psem[b]).wait()

                @pl.when(g > 0)
                def _():
                    pltpu.make_async_copy(
                        obuf_v.at[b], out.at[pl.ds(0, CH)], ssem[b]).wait()

                def row_body(r, c2):
                    for j in range(D // LANES):
                        sl = pl.ds(j * LANES, LANES)
                        obuf_v[b, r, sl] = (
                            rows_v[b, r, sl] * scale + pos_v[b, r, sl])
                    return c2

                lax.fori_loop(0, CH, row_body, 0)

                pltpu.async_copy(
                    obuf_v.at[b], out.at[pl.ds(base + ci * CH, CH)], ssem[b])

                @pl.when(ci + NBUF < n_ch)
                def _():
                    issue(ci + NBUF, b)
            return carry

        lax.fori_loop(0, n_g, group, 0)

        for b in range(NBUF):
            pltpu.make_async_copy(
                obuf_v.at[b], out.at[pl.ds(0, CH)], ssem[b]).wait()

    return ebd


def kernel(X, word_emb, pos_emb):
    B, L = X.shape
    V, D = word_emb.shape
    N = B * L
    info = plsc.get_sparse_core_info()
    ebd = _make_ebd(N, L, V, D, info.num_cores, info.num_subcores)
    Xf = X.reshape(N).astype(jnp.int32)
    out = ebd(word_emb, Xf, pos_emb[:L])
    return out.reshape(B, L, D)
